# baseline (speedup 1.0000x reference)
import functools

import jax
import jax.numpy as jnp
from jax import lax
from jax.experimental import pallas as pl
from jax.experimental.pallas import tpu as pltpu
from jax.experimental.pallas import tpu_sc as plsc

ROWS = 128
COLS = 8192
TOPK = 256
LANES = 16
NCORES = 2
NSUB = 16
NWORKERS = NCORES * NSUB
ROWS_PER_W = ROWS // NWORKERS
VECS = COLS // LANES
NBUCKETS = 256
NCHUNKS = NBUCKETS // LANES


def _pick_bucket(counts, k):
    cs = jnp.cumsum(counts)
    total = jnp.max(cs)
    ea = total - cs
    idx = jnp.sum((ea >= k).astype(jnp.int32))
    above = jnp.max(jnp.where(ea < k, ea, jnp.int32(0)))
    return idx, above


def _tec_body(x_hbm, out_hbm, rows_v, hist_v, h16_v, cand_v):
    wid = lax.axis_index("s") * NCORES + lax.axis_index("c")
    base = wid * ROWS_PER_W
    pltpu.sync_copy(x_hbm.at[pl.ds(base, ROWS_PER_W)], rows_v)

    iota = lax.iota(jnp.int32, LANES)
    ones = jnp.ones((LANES,), jnp.int32)
    zeros16 = jnp.zeros((LANES,), jnp.int32)

    def _select(k_rem):
        c0, above_chunks = _pick_bucket(h16_v[pl.ds(0, LANES)], k_rem)
        k2 = k_rem - above_chunks
        h = hist_v[pl.ds(c0 * LANES, LANES)]
        b0, above_in = _pick_bucket(h, k2)
        return c0 * LANES + b0, k2 - above_in

    def _zero_hist():
        for c in range(NCHUNKS):
            hist_v[pl.ds(c * LANES, LANES)] = zeros16
        h16_v[pl.ds(0, LANES)] = zeros16

    for r in range(ROWS_PER_W):
        _zero_hist()

        @plsc.parallel_loop(0, VECS, unroll=8)
        def _hist_scan(j, r=r):
            v = rows_v[r, pl.ds(j * LANES, LANES)]
            ab = lax.bitcast_convert_type(v, jnp.int32) & jnp.int32(0x7FFFFFFF)
            plsc.addupdate_scatter(
                hist_v, [lax.shift_right_logical(ab, 23)], ones)
            plsc.addupdate_scatter(
                h16_v, [lax.shift_right_logical(ab, 27)], ones)

        b1, k_rem = _select(jnp.int32(TOPK))
        prefix = b1

        @plsc.parallel_loop(0, VECS, unroll=4, carry=jnp.int32(0))
        def _compress(j, off, r=r, b1=b1):
            v = rows_v[r, pl.ds(j * LANES, LANES)]
            ab = lax.bitcast_convert_type(v, jnp.int32) & jnp.int32(0x7FFFFFFF)
            msk = lax.shift_right_logical(ab, 23) == b1
            plsc.store_compressed(cand_v.at[pl.ds(off, LANES)], ab, mask=msk)
            return off + plsc.all_reduce_population_count(msk)[0]

        ncand = _compress
        nvec = lax.shift_right_logical(ncand + (LANES - 1), 4)

        for shift, width in ((15, 8), (7, 8), (0, 7)):
            _zero_hist()
            top = shift + width

            @plsc.parallel_loop(0, nvec, unroll=2)
            def _cand_scan(j, shift=shift, top=top, width=width,
                           prefix=prefix, ncand=ncand):
                ab = cand_v[pl.ds(j * LANES, LANES)]
                msk = (j * LANES + iota) < ncand
                if top < 31:
                    msk = msk & (lax.shift_right_logical(ab, top) == prefix)
                digit = lax.shift_right_logical(ab, shift) & jnp.int32(
                    (1 << width) - 1)
                plsc.addupdate_scatter(hist_v, [digit], ones, mask=msk)
                plsc.addupdate_scatter(
                    h16_v, [lax.shift_right_logical(digit, 4)], ones, mask=msk)

            b0, k_rem = _select(k_rem)
            prefix = lax.shift_left(prefix, width) | b0

        thresh = prefix

        @plsc.parallel_loop(0, VECS, unroll=8)
        def _mask_scan(j, r=r, thresh=thresh):
            sl = pl.ds(j * LANES, LANES)
            v = rows_v[r, sl]
            ab = lax.bitcast_convert_type(v, jnp.int32) & jnp.int32(0x7FFFFFFF)
            rows_v[r, sl] = jnp.where(ab >= thresh, v, jnp.float32(0.0))

    pltpu.sync_copy(rows_v, out_hbm.at[pl.ds(base, ROWS_PER_W)])


_topk_call = functools.partial(
    pl.kernel,
    mesh=plsc.VectorSubcoreMesh(core_axis_name="c", subcore_axis_name="s"),
    out_type=jax.ShapeDtypeStruct((ROWS, COLS), jnp.float32),
    scratch_types=[
        pltpu.VMEM((ROWS_PER_W, COLS), jnp.float32),
        pltpu.VMEM((NBUCKETS,), jnp.int32),
        pltpu.VMEM((LANES,), jnp.int32),
        pltpu.VMEM((COLS + LANES,), jnp.int32),
    ],
    compiler_params=pltpu.CompilerParams(needs_layout_passes=False),
)(_tec_body)


@jax.jit
def kernel(inputs):
    return _topk_call(inputs)


# speedup vs baseline: 1.3776x; 1.3776x over previous
import functools

import jax
import jax.numpy as jnp
from jax import lax
from jax.experimental import pallas as pl
from jax.experimental.pallas import tpu as pltpu
from jax.experimental.pallas import tpu_sc as plsc

ROWS = 128
COLS = 8192
TOPK = 256
LANES = 16
NCORES = 2
NSUB = 16
NWORKERS = NCORES * NSUB
ROWS_PER_W = ROWS // NWORKERS
VECS = COLS // LANES
NBUCKETS = 256
NCHUNKS = NBUCKETS // LANES
SPREAD = 4


def _pick_bucket(counts, k):
    cs = jnp.cumsum(counts)
    total = jnp.max(cs)
    ea = total - cs
    idx = jnp.sum((ea >= k).astype(jnp.int32))
    above = jnp.max(jnp.where(ea < k, ea, jnp.int32(0)))
    return idx, above


def _tec_body(x_hbm, out_hbm, rows_v, hist_v, hsp_v, cand_v):
    wid = lax.axis_index("s") * NCORES + lax.axis_index("c")
    base = wid * ROWS_PER_W
    pltpu.sync_copy(x_hbm.at[pl.ds(base, ROWS_PER_W)], rows_v)

    iota = lax.iota(jnp.int32, LANES)
    ones = jnp.ones((LANES,), jnp.int32)
    zeros16 = jnp.zeros((LANES,), jnp.int32)

    def _select(k_rem):
        chunk_sums = zeros16
        for c in range(NCHUNKS):
            s = jnp.sum(hist_v[pl.ds(c * LANES, LANES)])
            chunk_sums = chunk_sums + jnp.where(iota == c, s, jnp.int32(0))
        c0, above_chunks = _pick_bucket(chunk_sums, k_rem)
        k2 = k_rem - above_chunks
        h = hist_v[pl.ds(c0 * LANES, LANES)]
        b0, above_in = _pick_bucket(h, k2)
        return c0 * LANES + b0, k2 - above_in

    def _select_spread(k_rem):
        chunk_sums = zeros16
        for c in range(NCHUNKS):
            acc = hsp_v[pl.ds(c * LANES * SPREAD, LANES)]
            for t in range(1, SPREAD):
                acc = acc + hsp_v[pl.ds(c * LANES * SPREAD + t * LANES, LANES)]
            s = jnp.sum(acc)
            chunk_sums = chunk_sums + jnp.where(iota == c, s, jnp.int32(0))
        c0, above_chunks = _pick_bucket(chunk_sums, k_rem)
        k2 = k_rem - above_chunks
        fbase = c0 * (LANES * SPREAD) + iota * SPREAD
        fine = plsc.load_gather(hsp_v, [fbase])
        for t in range(1, SPREAD):
            fine = fine + plsc.load_gather(hsp_v, [fbase + t])
        b0, above_in = _pick_bucket(fine, k2)
        return c0 * LANES + b0, k2 - above_in

    def _zero_hist():
        for c in range(NCHUNKS):
            hist_v[pl.ds(c * LANES, LANES)] = zeros16

    lane_sp = iota & jnp.int32(SPREAD - 1)

    for r in range(ROWS_PER_W):
        @plsc.parallel_loop(0, NBUCKETS * SPREAD // LANES, unroll=8)
        def _zsp(c):
            hsp_v[pl.ds(c * LANES, LANES)] = zeros16

        @plsc.parallel_loop(0, VECS, unroll=8)
        def _hist_scan(j, r=r):
            v = rows_v[r, pl.ds(j * LANES, LANES)]
            ab = lax.bitcast_convert_type(v, jnp.int32) & jnp.int32(0x7FFFFFFF)
            d = lax.shift_right_logical(ab, 23)
            plsc.addupdate_scatter(
                hsp_v, [d * SPREAD + lane_sp], ones)

        b1, k_rem = _select_spread(jnp.int32(TOPK))
        prefix = b1

        @plsc.parallel_loop(0, VECS, unroll=4, carry=jnp.int32(0))
        def _compress(j, off, r=r, b1=b1):
            v = rows_v[r, pl.ds(j * LANES, LANES)]
            ab = lax.bitcast_convert_type(v, jnp.int32) & jnp.int32(0x7FFFFFFF)
            msk = lax.shift_right_logical(ab, 23) == b1
            plsc.store_compressed(cand_v.at[pl.ds(off, LANES)], ab, mask=msk)
            return off + plsc.all_reduce_population_count(msk)[0]

        ncand = _compress
        nvec = lax.shift_right_logical(ncand + (LANES - 1), 4)

        for shift, width in ((15, 8), (7, 8), (0, 7)):
            _zero_hist()
            top = shift + width

            @plsc.parallel_loop(0, nvec, unroll=2)
            def _cand_scan(j, shift=shift, top=top, width=width,
                           prefix=prefix, ncand=ncand):
                ab = cand_v[pl.ds(j * LANES, LANES)]
                msk = (j * LANES + iota) < ncand
                if top < 31:
                    msk = msk & (lax.shift_right_logical(ab, top) == prefix)
                digit = lax.shift_right_logical(ab, shift) & jnp.int32(
                    (1 << width) - 1)
                plsc.addupdate_scatter(hist_v, [digit], ones, mask=msk)

            b0, k_rem = _select(k_rem)
            prefix = lax.shift_left(prefix, width) | b0

        thresh = prefix

        @plsc.parallel_loop(0, VECS, unroll=8)
        def _mask_scan(j, r=r, thresh=thresh):
            sl = pl.ds(j * LANES, LANES)
            v = rows_v[r, sl]
            ab = lax.bitcast_convert_type(v, jnp.int32) & jnp.int32(0x7FFFFFFF)
            rows_v[r, sl] = jnp.where(ab >= thresh, v, jnp.float32(0.0))

    pltpu.sync_copy(rows_v, out_hbm.at[pl.ds(base, ROWS_PER_W)])


_topk_call = functools.partial(
    pl.kernel,
    mesh=plsc.VectorSubcoreMesh(core_axis_name="c", subcore_axis_name="s"),
    out_type=jax.ShapeDtypeStruct((ROWS, COLS), jnp.float32),
    scratch_types=[
        pltpu.VMEM((ROWS_PER_W, COLS), jnp.float32),
        pltpu.VMEM((NBUCKETS,), jnp.int32),
        pltpu.VMEM((NBUCKETS * SPREAD,), jnp.int32),
        pltpu.VMEM((COLS + LANES,), jnp.int32),
    ],
    compiler_params=pltpu.CompilerParams(needs_layout_passes=False),
)(_tec_body)


@jax.jit
def kernel(inputs):
    return _topk_call(inputs)


# speedup vs baseline: 1.3949x; 1.0126x over previous
import functools

import jax
import jax.numpy as jnp
from jax import lax
from jax.experimental import pallas as pl
from jax.experimental.pallas import tpu as pltpu
from jax.experimental.pallas import tpu_sc as plsc

ROWS = 128
COLS = 8192
TOPK = 256
LANES = 16
NCORES = 2
NSUB = 16
NWORKERS = NCORES * NSUB
ROWS_PER_W = ROWS // NWORKERS
VECS = COLS // LANES
NBUCKETS = 256
NCHUNKS = NBUCKETS // LANES
SPREAD = 4


def _pick_bucket(counts, k):
    cs = jnp.cumsum(counts)
    total = jnp.max(cs)
    ea = total - cs
    idx = jnp.sum((ea >= k).astype(jnp.int32))
    above = jnp.max(jnp.where(ea < k, ea, jnp.int32(0)))
    return idx, above


def _tec_body(x_hbm, out_hbm, rows_v, hist_v, hsp_v, cand_v, insem, outsem):
    wid = lax.axis_index("s") * NCORES + lax.axis_index("c")
    base = wid * ROWS_PER_W
    in_descs = [
        pltpu.async_copy(x_hbm.at[base + r], rows_v.at[r], insem.at[r])
        for r in range(ROWS_PER_W)
    ]
    out_descs = []

    iota = lax.iota(jnp.int32, LANES)
    ones = jnp.ones((LANES,), jnp.int32)
    zeros16 = jnp.zeros((LANES,), jnp.int32)

    def _select(k_rem):
        chunk_sums = zeros16
        for c in range(NCHUNKS):
            s = jnp.sum(hist_v[pl.ds(c * LANES, LANES)])
            chunk_sums = chunk_sums + jnp.where(iota == c, s, jnp.int32(0))
        c0, above_chunks = _pick_bucket(chunk_sums, k_rem)
        k2 = k_rem - above_chunks
        h = hist_v[pl.ds(c0 * LANES, LANES)]
        b0, above_in = _pick_bucket(h, k2)
        return c0 * LANES + b0, k2 - above_in

    def _select_spread(k_rem):
        chunk_sums = zeros16
        for c in range(NCHUNKS):
            acc = hsp_v[pl.ds(c * LANES * SPREAD, LANES)]
            for t in range(1, SPREAD):
                acc = acc + hsp_v[pl.ds(c * LANES * SPREAD + t * LANES, LANES)]
            s = jnp.sum(acc)
            chunk_sums = chunk_sums + jnp.where(iota == c, s, jnp.int32(0))
        c0, above_chunks = _pick_bucket(chunk_sums, k_rem)
        k2 = k_rem - above_chunks
        fbase = c0 * (LANES * SPREAD) + iota * SPREAD
        fine = plsc.load_gather(hsp_v, [fbase])
        for t in range(1, SPREAD):
            fine = fine + plsc.load_gather(hsp_v, [fbase + t])
        b0, above_in = _pick_bucket(fine, k2)
        return c0 * LANES + b0, k2 - above_in

    def _zero_hist():
        for c in range(NCHUNKS):
            hist_v[pl.ds(c * LANES, LANES)] = zeros16

    lane_sp = iota & jnp.int32(SPREAD - 1)

    for r in range(ROWS_PER_W):
        in_descs[r].wait()
        @plsc.parallel_loop(0, NBUCKETS * SPREAD // LANES, unroll=8)
        def _zsp(c):
            hsp_v[pl.ds(c * LANES, LANES)] = zeros16

        @plsc.parallel_loop(0, VECS, unroll=8)
        def _hist_scan(j, r=r):
            v = rows_v[r, pl.ds(j * LANES, LANES)]
            ab = lax.bitcast_convert_type(v, jnp.int32) & jnp.int32(0x7FFFFFFF)
            d = lax.shift_right_logical(ab, 23)
            plsc.addupdate_scatter(
                hsp_v, [d * SPREAD + lane_sp], ones)

        b1, k_rem = _select_spread(jnp.int32(TOPK))
        prefix = b1

        @plsc.parallel_loop(0, VECS, unroll=4, carry=jnp.int32(0))
        def _compress(j, off, r=r, b1=b1):
            v = rows_v[r, pl.ds(j * LANES, LANES)]
            ab = lax.bitcast_convert_type(v, jnp.int32) & jnp.int32(0x7FFFFFFF)
            msk = lax.shift_right_logical(ab, 23) == b1
            plsc.store_compressed(cand_v.at[pl.ds(off, LANES)], ab, mask=msk)
            return off + plsc.all_reduce_population_count(msk)[0]

        ncand = _compress
        nvec = lax.shift_right_logical(ncand + (LANES - 1), 4)

        for shift, width in ((15, 8), (7, 8), (0, 7)):
            _zero_hist()
            top = shift + width

            @plsc.parallel_loop(0, nvec, unroll=2)
            def _cand_scan(j, shift=shift, top=top, width=width,
                           prefix=prefix, ncand=ncand):
                ab = cand_v[pl.ds(j * LANES, LANES)]
                msk = (j * LANES + iota) < ncand
                if top < 31:
                    msk = msk & (lax.shift_right_logical(ab, top) == prefix)
                digit = lax.shift_right_logical(ab, shift) & jnp.int32(
                    (1 << width) - 1)
                plsc.addupdate_scatter(hist_v, [digit], ones, mask=msk)

            b0, k_rem = _select(k_rem)
            prefix = lax.shift_left(prefix, width) | b0

        thresh = prefix

        @plsc.parallel_loop(0, VECS, unroll=8)
        def _mask_scan(j, r=r, thresh=thresh):
            sl = pl.ds(j * LANES, LANES)
            v = rows_v[r, sl]
            ab = lax.bitcast_convert_type(v, jnp.int32) & jnp.int32(0x7FFFFFFF)
            rows_v[r, sl] = jnp.where(ab >= thresh, v, jnp.float32(0.0))

        out_descs.append(
            pltpu.async_copy(rows_v.at[r], out_hbm.at[base + r], outsem.at[r]))

    for d in out_descs:
        d.wait()


_topk_call = functools.partial(
    pl.kernel,
    mesh=plsc.VectorSubcoreMesh(core_axis_name="c", subcore_axis_name="s"),
    out_type=jax.ShapeDtypeStruct((ROWS, COLS), jnp.float32),
    scratch_types=[
        pltpu.VMEM((ROWS_PER_W, COLS), jnp.float32),
        pltpu.VMEM((NBUCKETS,), jnp.int32),
        pltpu.VMEM((NBUCKETS * SPREAD,), jnp.int32),
        pltpu.VMEM((COLS + LANES,), jnp.int32),
        pltpu.SemaphoreType.DMA((ROWS_PER_W,)),
        pltpu.SemaphoreType.DMA((ROWS_PER_W,)),
    ],
    compiler_params=pltpu.CompilerParams(needs_layout_passes=False),
)(_tec_body)


@jax.jit
def kernel(inputs):
    return _topk_call(inputs)


# speedup vs baseline: 1.4004x; 1.0039x over previous
import functools

import jax
import jax.numpy as jnp
from jax import lax
from jax.experimental import pallas as pl
from jax.experimental.pallas import tpu as pltpu
from jax.experimental.pallas import tpu_sc as plsc

ROWS = 128
COLS = 8192
TOPK = 256
LANES = 16
NCORES = 2
NSUB = 16
NWORKERS = NCORES * NSUB
ROWS_PER_W = ROWS // NWORKERS
VECS = COLS // LANES
NBUCKETS = 256
NCHUNKS = NBUCKETS // LANES
SPREAD = 4


def _pick_bucket(counts, k):
    cs = jnp.cumsum(counts)
    total = jnp.max(cs)
    ea = total - cs
    idx = jnp.sum((ea >= k).astype(jnp.int32))
    above = jnp.max(jnp.where(ea < k, ea, jnp.int32(0)))
    return idx, above


def _tec_body(x_hbm, out_hbm, rows_v, hist_v, hsp_v, cand_v, insem, outsem):
    wid = lax.axis_index("s") * NCORES + lax.axis_index("c")
    base = wid * ROWS_PER_W
    in_descs = [
        pltpu.async_copy(x_hbm.at[base + r], rows_v.at[r], insem.at[r])
        for r in range(ROWS_PER_W)
    ]
    out_descs = []

    iota = lax.iota(jnp.int32, LANES)
    ones = jnp.ones((LANES,), jnp.int32)
    zeros16 = jnp.zeros((LANES,), jnp.int32)

    def _select(k_rem):
        chunk_sums = zeros16
        for c in range(NCHUNKS):
            s = jnp.sum(hist_v[pl.ds(c * LANES, LANES)])
            chunk_sums = chunk_sums + jnp.where(iota == c, s, jnp.int32(0))
        c0, above_chunks = _pick_bucket(chunk_sums, k_rem)
        k2 = k_rem - above_chunks
        h = hist_v[pl.ds(c0 * LANES, LANES)]
        b0, above_in = _pick_bucket(h, k2)
        return c0 * LANES + b0, k2 - above_in

    def _select_spread(k_rem):
        chunk_sums = zeros16
        for c in range(NCHUNKS):
            acc = hsp_v[pl.ds(c * LANES * SPREAD, LANES)]
            for t in range(1, SPREAD):
                acc = acc + hsp_v[pl.ds(c * LANES * SPREAD + t * LANES, LANES)]
            s = jnp.sum(acc)
            chunk_sums = chunk_sums + jnp.where(iota == c, s, jnp.int32(0))
        c0, above_chunks = _pick_bucket(chunk_sums, k_rem)
        k2 = k_rem - above_chunks
        fbase = c0 * (LANES * SPREAD) + iota * SPREAD
        fine = plsc.load_gather(hsp_v, [fbase])
        for t in range(1, SPREAD):
            fine = fine + plsc.load_gather(hsp_v, [fbase + t])
        b0, above_in = _pick_bucket(fine, k2)
        return c0 * LANES + b0, k2 - above_in

    def _zero_hist():
        for c in range(NCHUNKS):
            hist_v[pl.ds(c * LANES, LANES)] = zeros16

    lane_sp = iota & jnp.int32(SPREAD - 1)

    for r in range(ROWS_PER_W):
        in_descs[r].wait()
        @plsc.parallel_loop(0, NBUCKETS * SPREAD // LANES, unroll=8)
        def _zsp(c):
            hsp_v[pl.ds(c * LANES, LANES)] = zeros16

        @plsc.parallel_loop(0, VECS, unroll=8)
        def _hist_scan(j, r=r):
            v = rows_v[r, pl.ds(j * LANES, LANES)]
            ab = lax.bitcast_convert_type(v, jnp.int32) & jnp.int32(0x7FFFFFFF)
            d = lax.shift_right_logical(ab, 23)
            plsc.addupdate_scatter(
                hsp_v, [d * SPREAD + lane_sp], ones)

        b1, k_rem = _select_spread(jnp.int32(TOPK))
        prefix = b1

        @plsc.parallel_loop(0, VECS, step=4, unroll=2, carry=jnp.int32(0))
        def _compress(j, off, r=r, b1=b1):
            abs_, msks, psum = [], [], []
            acc = None
            for t in range(4):
                v = rows_v[r, pl.ds((j + t) * LANES, LANES)]
                ab = lax.bitcast_convert_type(v, jnp.int32) & jnp.int32(
                    0x7FFFFFFF)
                m = lax.shift_right_logical(ab, 23) == b1
                abs_.append(ab)
                msks.append(m)
                p = plsc.all_reduce_population_count(m)
                acc = p if acc is None else acc + p
                psum.append(acc)
            plsc.store_compressed(cand_v.at[pl.ds(off, LANES)],
                                  abs_[0], mask=msks[0])
            for t in range(1, 4):
                plsc.store_compressed(
                    cand_v.at[pl.ds(off + psum[t - 1][0], LANES)],
                    abs_[t], mask=msks[t])
            return off + psum[3][0]

        ncand = _compress
        nvec = lax.shift_right_logical(ncand + (LANES - 1), 4)

        for shift, width in ((15, 8), (7, 8), (0, 7)):
            _zero_hist()
            top = shift + width

            @plsc.parallel_loop(0, nvec, unroll=2)
            def _cand_scan(j, shift=shift, top=top, width=width,
                           prefix=prefix, ncand=ncand):
                ab = cand_v[pl.ds(j * LANES, LANES)]
                msk = (j * LANES + iota) < ncand
                if top < 31:
                    msk = msk & (lax.shift_right_logical(ab, top) == prefix)
                digit = lax.shift_right_logical(ab, shift) & jnp.int32(
                    (1 << width) - 1)
                plsc.addupdate_scatter(hist_v, [digit], ones, mask=msk)

            b0, k_rem = _select(k_rem)
            prefix = lax.shift_left(prefix, width) | b0

        thresh = prefix

        @plsc.parallel_loop(0, VECS, unroll=8)
        def _mask_scan(j, r=r, thresh=thresh):
            sl = pl.ds(j * LANES, LANES)
            v = rows_v[r, sl]
            ab = lax.bitcast_convert_type(v, jnp.int32) & jnp.int32(0x7FFFFFFF)
            rows_v[r, sl] = jnp.where(ab >= thresh, v, jnp.float32(0.0))

        out_descs.append(
            pltpu.async_copy(rows_v.at[r], out_hbm.at[base + r], outsem.at[r]))

    for d in out_descs:
        d.wait()


_topk_call = functools.partial(
    pl.kernel,
    mesh=plsc.VectorSubcoreMesh(core_axis_name="c", subcore_axis_name="s"),
    out_type=jax.ShapeDtypeStruct((ROWS, COLS), jnp.float32),
    scratch_types=[
        pltpu.VMEM((ROWS_PER_W, COLS), jnp.float32),
        pltpu.VMEM((NBUCKETS,), jnp.int32),
        pltpu.VMEM((NBUCKETS * SPREAD,), jnp.int32),
        pltpu.VMEM((COLS + LANES,), jnp.int32),
        pltpu.SemaphoreType.DMA((ROWS_PER_W,)),
        pltpu.SemaphoreType.DMA((ROWS_PER_W,)),
    ],
    compiler_params=pltpu.CompilerParams(needs_layout_passes=False),
)(_tec_body)


@jax.jit
def kernel(inputs):
    return _topk_call(inputs)


# speedup vs baseline: 1.4487x; 1.0344x over previous
import functools

import jax
import jax.numpy as jnp
from jax import lax
from jax.experimental import pallas as pl
from jax.experimental.pallas import tpu as pltpu
from jax.experimental.pallas import tpu_sc as plsc

ROWS = 128
COLS = 8192
TOPK = 256
LANES = 16
NCORES = 2
NSUB = 16
NWORKERS = NCORES * NSUB
ROWS_PER_W = ROWS // NWORKERS
VECS = COLS // LANES
NBUCKETS = 256
NCHUNKS = NBUCKETS // LANES
SPREAD = 4


def _pick_bucket(counts, k):
    cs = jnp.cumsum(counts)
    total = jnp.max(cs)
    ea = total - cs
    idx = jnp.sum((ea >= k).astype(jnp.int32))
    above = jnp.max(jnp.where(ea < k, ea, jnp.int32(0)))
    return idx, above


def _tec_body(x_hbm, out_hbm, rows_v, hist_v, hsp_v, h16_v, cand_v, insem, outsem):
    wid = lax.axis_index("s") * NCORES + lax.axis_index("c")
    base = wid * ROWS_PER_W
    in_descs = [
        pltpu.async_copy(x_hbm.at[base + r], rows_v.at[r], insem.at[r])
        for r in range(ROWS_PER_W)
    ]
    out_descs = []

    iota = lax.iota(jnp.int32, LANES)
    ones = jnp.ones((LANES,), jnp.int32)
    zeros16 = jnp.zeros((LANES,), jnp.int32)

    def _select(k_rem):
        c0, above_chunks = _pick_bucket(h16_v[pl.ds(0, LANES)], k_rem)
        k2 = k_rem - above_chunks
        h = hist_v[pl.ds(c0 * LANES, LANES)]
        b0, above_in = _pick_bucket(h, k2)
        return c0 * LANES + b0, k2 - above_in

    def _select_spread(k_rem):
        chunk_sums = zeros16
        for c in range(NCHUNKS):
            acc = hsp_v[pl.ds(c * LANES * SPREAD, LANES)]
            for t in range(1, SPREAD):
                acc = acc + hsp_v[pl.ds(c * LANES * SPREAD + t * LANES, LANES)]
            s = jnp.sum(acc)
            chunk_sums = chunk_sums + jnp.where(iota == c, s, jnp.int32(0))
        c0, above_chunks = _pick_bucket(chunk_sums, k_rem)
        k2 = k_rem - above_chunks
        fbase = c0 * (LANES * SPREAD) + iota * SPREAD
        fine = plsc.load_gather(hsp_v, [fbase])
        for t in range(1, SPREAD):
            fine = fine + plsc.load_gather(hsp_v, [fbase + t])
        b0, above_in = _pick_bucket(fine, k2)
        return c0 * LANES + b0, k2 - above_in

    def _zero_hist():
        for c in range(NCHUNKS):
            hist_v[pl.ds(c * LANES, LANES)] = zeros16
        h16_v[pl.ds(0, LANES)] = zeros16

    lane_sp = iota & jnp.int32(SPREAD - 1)

    for r in range(ROWS_PER_W):
        in_descs[r].wait()
        @plsc.parallel_loop(0, NBUCKETS * SPREAD // LANES, unroll=8)
        def _zsp(c):
            hsp_v[pl.ds(c * LANES, LANES)] = zeros16

        @plsc.parallel_loop(0, VECS, unroll=8)
        def _hist_scan(j, r=r):
            v = rows_v[r, pl.ds(j * LANES, LANES)]
            ab = lax.bitcast_convert_type(v, jnp.int32) & jnp.int32(0x7FFFFFFF)
            d = lax.shift_right_logical(ab, 23)
            plsc.addupdate_scatter(
                hsp_v, [d * SPREAD + lane_sp], ones)

        b1, k_rem = _select_spread(jnp.int32(TOPK))
        prefix = b1

        @plsc.parallel_loop(0, VECS, step=4, unroll=2, carry=jnp.int32(0))
        def _compress(j, off, r=r, b1=b1):
            abs_, msks, psum = [], [], []
            acc = None
            for t in range(4):
                v = rows_v[r, pl.ds((j + t) * LANES, LANES)]
                ab = lax.bitcast_convert_type(v, jnp.int32) & jnp.int32(
                    0x7FFFFFFF)
                m = lax.shift_right_logical(ab, 23) == b1
                abs_.append(ab)
                msks.append(m)
                p = plsc.all_reduce_population_count(m)
                acc = p if acc is None else acc + p
                psum.append(acc)
            plsc.store_compressed(cand_v.at[pl.ds(off, LANES)],
                                  abs_[0], mask=msks[0])
            for t in range(1, 4):
                plsc.store_compressed(
                    cand_v.at[pl.ds(off + psum[t - 1][0], LANES)],
                    abs_[t], mask=msks[t])
            return off + psum[3][0]

        ncand = _compress
        nvec = lax.shift_right_logical(ncand + (LANES - 1), 4)

        for shift, width in ((15, 8), (7, 8), (0, 7)):
            _zero_hist()
            top = shift + width

            @plsc.parallel_loop(0, nvec, unroll=2)
            def _cand_scan(j, shift=shift, top=top, width=width,
                           prefix=prefix, ncand=ncand):
                ab = cand_v[pl.ds(j * LANES, LANES)]
                msk = (j * LANES + iota) < ncand
                if top < 31:
                    msk = msk & (lax.shift_right_logical(ab, top) == prefix)
                digit = lax.shift_right_logical(ab, shift) & jnp.int32(
                    (1 << width) - 1)
                plsc.addupdate_scatter(hist_v, [digit], ones, mask=msk)
                plsc.addupdate_scatter(
                    h16_v, [lax.shift_right_logical(digit, 4)], ones,
                    mask=msk)

            b0, k_rem = _select(k_rem)
            prefix = lax.shift_left(prefix, width) | b0

        thresh = prefix

        @plsc.parallel_loop(0, VECS, unroll=8)
        def _mask_scan(j, r=r, thresh=thresh):
            sl = pl.ds(j * LANES, LANES)
            v = rows_v[r, sl]
            ab = lax.bitcast_convert_type(v, jnp.int32) & jnp.int32(0x7FFFFFFF)
            rows_v[r, sl] = jnp.where(ab >= thresh, v, jnp.float32(0.0))

        out_descs.append(
            pltpu.async_copy(rows_v.at[r], out_hbm.at[base + r], outsem.at[r]))

    for d in out_descs:
        d.wait()


_topk_call = functools.partial(
    pl.kernel,
    mesh=plsc.VectorSubcoreMesh(core_axis_name="c", subcore_axis_name="s"),
    out_type=jax.ShapeDtypeStruct((ROWS, COLS), jnp.float32),
    scratch_types=[
        pltpu.VMEM((ROWS_PER_W, COLS), jnp.float32),
        pltpu.VMEM((NBUCKETS,), jnp.int32),
        pltpu.VMEM((NBUCKETS * SPREAD,), jnp.int32),
        pltpu.VMEM((LANES,), jnp.int32),
        pltpu.VMEM((COLS + LANES,), jnp.int32),
        pltpu.SemaphoreType.DMA((ROWS_PER_W,)),
        pltpu.SemaphoreType.DMA((ROWS_PER_W,)),
    ],
    compiler_params=pltpu.CompilerParams(needs_layout_passes=False),
)(_tec_body)


@jax.jit
def kernel(inputs):
    return _topk_call(inputs)
